# Initial kernel scaffold; baseline (speedup 1.0000x reference)
import jax
import jax.numpy as jnp
from jax.experimental import pallas as pl


def kernel(x, proj_w1, proj_b1, proj_w2, proj_b2, query, ln_q_g, ln_q_b, ln_kv_g, ln_kv_b, in_proj_w, in_proj_b, out_proj_w, out_proj_b, w_gate, pos_embed):
    raise NotImplementedError("write your pallas kernel here")


# speedup vs baseline: 1.8671x; 1.8671x over previous
import functools

import jax
import jax.numpy as jnp
from jax.experimental import pallas as pl

MM = 1024
HID = 2048
N_Q = 576
HEADS = 8
HEAD_DIM = 128
BATCH = 4
TOKENS = BATCH * N_Q

_DOT = functools.partial(jnp.dot, preferred_element_type=jnp.float32)


def _ln(x, g, b, eps=1e-5):
    mu = jnp.mean(x, axis=-1, keepdims=True)
    var = jnp.mean((x - mu) ** 2, axis=-1, keepdims=True)
    return (x - mu) * jax.lax.rsqrt(var + eps) * g + b


def _q_kernel(query_ref, pos_ref, g_ref, b_ref, wqt_ref, bq_ref, qh_ref):
    qf = _ln(query_ref[...], g_ref[...], b_ref[...]) + pos_ref[...]
    qh_ref[...] = _DOT(qf, wqt_ref[...]) + bq_ref[...]


def _kv_kernel(x_ref, pos_ref, g_ref, b_ref, wkt_ref, wvt_ref, bk_ref, bv_ref,
               kh_ref, vh_ref):
    kv = _ln(x_ref[...], g_ref[...], b_ref[...])
    kh_ref[...] = _DOT(kv + pos_ref[...], wkt_ref[...]) + bk_ref[...]
    vh_ref[...] = _DOT(kv, wvt_ref[...]) + bv_ref[...]


def _attn_kernel(qh_ref, kh_ref, vh_ref, o_ref):
    q = qh_ref[...]
    k = kh_ref[...]
    v = vh_ref[...]
    s = jax.lax.dot_general(q, k, (((1,), (1,)), ((), ())),
                            preferred_element_type=jnp.float32)
    s = s * (1.0 / (HEAD_DIM ** 0.5))
    s = s - jnp.max(s, axis=-1, keepdims=True)
    e = jnp.exp(s)
    p = e / jnp.sum(e, axis=-1, keepdims=True)
    o_ref[...] = _DOT(p, v)


def _mm_bias_kernel(x_ref, wt_ref, b_ref, o_ref):
    o_ref[...] = _DOT(x_ref[...], wt_ref[...]) + b_ref[...]


def _gates_kernel(x_ref, wgt_ref, g_ref):
    x = x_ref[...]
    wgt = wgt_ref[...]
    l0 = jnp.sum(x * wgt[0:1, :], axis=1, keepdims=True)
    l1 = jnp.sum(x * wgt[1:2, :], axis=1, keepdims=True)
    m = jnp.maximum(l0, l1)
    e0 = jnp.exp(l0 - m)
    e1 = jnp.exp(l1 - m)
    s = e0 + e1
    p0 = e0 / s
    p1 = e1 / s
    denom = p0 + p1 + 1e-6
    g_ref[...] = jnp.concatenate([p0 / denom, p1 / denom], axis=1)


def _gelu(h):
    return 0.5 * h * (1.0 + jax.lax.erf(h * 0.7071067811865476))


def _h_kernel(x_ref, a_ref, w1_ref, b1_ref, g_ref, hm_ref):
    g = g_ref[...]
    g0 = g[:, 0:1]
    g1 = g[:, 1:2]
    h0 = _gelu(_DOT(x_ref[...], w1_ref[...]) + b1_ref[...])
    h1 = _gelu(_DOT(a_ref[...], w1_ref[...]) + b1_ref[...])
    hm_ref[...] = g0 * h0 + g1 * h1


def _out_kernel(hm_ref, w2_ref, b2_ref, g_ref, o_ref):
    g = g_ref[...]
    gsum = g[:, 0:1] + g[:, 1:2]
    o_ref[...] = _DOT(hm_ref[...], w2_ref[...]) + gsum * b2_ref[...]


def kernel(x, proj_w1, proj_b1, proj_w2, proj_b2, query, ln_q_g, ln_q_b,
           ln_kv_g, ln_kv_b, in_proj_w, in_proj_b, out_proj_w, out_proj_b,
           w_gate, pos_embed):
    f32 = jnp.float32
    xf = x.reshape(TOKENS, MM)
    wq, wk, wv = in_proj_w[:MM], in_proj_w[MM:2 * MM], in_proj_w[2 * MM:]
    bq, bk, bv = (in_proj_b[:MM][None, :], in_proj_b[MM:2 * MM][None, :],
                  in_proj_b[2 * MM:][None, :])
    wqt, wkt, wvt = wq.T, wk.T, wv.T
    owt = out_proj_w.T
    ln_q_g2, ln_q_b2 = ln_q_g[None, :], ln_q_b[None, :]
    ln_kv_g2, ln_kv_b2 = ln_kv_g[None, :], ln_kv_b[None, :]
    b1_2 = proj_b1[None, :]
    b2_2 = proj_b2[None, :]
    ob_2 = out_proj_b[None, :]
    wgt = w_gate.T

    qh = pl.pallas_call(
        _q_kernel,
        out_shape=jax.ShapeDtypeStruct((N_Q, MM), f32),
    )(query, pos_embed, ln_q_g2, ln_q_b2, wqt, bq)

    kh, vh = pl.pallas_call(
        _kv_kernel,
        grid=(BATCH,),
        in_specs=[
            pl.BlockSpec((N_Q, MM), lambda i: (i, 0)),
            pl.BlockSpec((N_Q, MM), lambda i: (0, 0)),
            pl.BlockSpec((1, MM), lambda i: (0, 0)),
            pl.BlockSpec((1, MM), lambda i: (0, 0)),
            pl.BlockSpec((MM, MM), lambda i: (0, 0)),
            pl.BlockSpec((MM, MM), lambda i: (0, 0)),
            pl.BlockSpec((1, MM), lambda i: (0, 0)),
            pl.BlockSpec((1, MM), lambda i: (0, 0)),
        ],
        out_specs=[
            pl.BlockSpec((N_Q, MM), lambda i: (i, 0)),
            pl.BlockSpec((N_Q, MM), lambda i: (i, 0)),
        ],
        out_shape=[
            jax.ShapeDtypeStruct((TOKENS, MM), f32),
            jax.ShapeDtypeStruct((TOKENS, MM), f32),
        ],
    )(xf, pos_embed, ln_kv_g2, ln_kv_b2, wkt, wvt, bk, bv)

    ao = pl.pallas_call(
        _attn_kernel,
        grid=(BATCH, HEADS),
        in_specs=[
            pl.BlockSpec((N_Q, HEAD_DIM), lambda b, h: (0, h)),
            pl.BlockSpec((N_Q, HEAD_DIM), lambda b, h: (b, h)),
            pl.BlockSpec((N_Q, HEAD_DIM), lambda b, h: (b, h)),
        ],
        out_specs=pl.BlockSpec((N_Q, HEAD_DIM), lambda b, h: (b, h)),
        out_shape=jax.ShapeDtypeStruct((TOKENS, MM), f32),
    )(qh, kh, vh)

    MT = 256
    attn_out = pl.pallas_call(
        _mm_bias_kernel,
        grid=(TOKENS // MT,),
        in_specs=[
            pl.BlockSpec((MT, MM), lambda i: (i, 0)),
            pl.BlockSpec((MM, MM), lambda i: (0, 0)),
            pl.BlockSpec((1, MM), lambda i: (0, 0)),
        ],
        out_specs=pl.BlockSpec((MT, MM), lambda i: (i, 0)),
        out_shape=jax.ShapeDtypeStruct((TOKENS, MM), f32),
    )(ao, owt, ob_2)

    gates = pl.pallas_call(
        _gates_kernel,
        out_shape=jax.ShapeDtypeStruct((TOKENS, 2), f32),
    )(xf, wgt)

    NT = 512
    hm = pl.pallas_call(
        _h_kernel,
        grid=(TOKENS // MT, HID // NT),
        in_specs=[
            pl.BlockSpec((MT, MM), lambda i, j: (i, 0)),
            pl.BlockSpec((MT, MM), lambda i, j: (i, 0)),
            pl.BlockSpec((MM, NT), lambda i, j: (0, j)),
            pl.BlockSpec((1, NT), lambda i, j: (0, j)),
            pl.BlockSpec((MT, 2), lambda i, j: (i, 0)),
        ],
        out_specs=pl.BlockSpec((MT, NT), lambda i, j: (i, j)),
        out_shape=jax.ShapeDtypeStruct((TOKENS, HID), f32),
    )(xf, attn_out, proj_w1, b1_2, gates)

    out = pl.pallas_call(
        _out_kernel,
        grid=(TOKENS // MT, HID // NT),
        in_specs=[
            pl.BlockSpec((MT, HID), lambda i, j: (i, 0)),
            pl.BlockSpec((HID, NT), lambda i, j: (0, j)),
            pl.BlockSpec((1, NT), lambda i, j: (0, j)),
            pl.BlockSpec((MT, 2), lambda i, j: (i, 0)),
        ],
        out_specs=pl.BlockSpec((MT, NT), lambda i, j: (i, j)),
        out_shape=jax.ShapeDtypeStruct((TOKENS, HID), f32),
    )(hm, proj_w2, b2_2, gates)

    return out.reshape(BATCH, N_Q, HID)


# speedup vs baseline: 3.8064x; 2.0386x over previous
import functools

import jax
import jax.numpy as jnp
from jax.experimental import pallas as pl
from jax.experimental.pallas import tpu as pltpu

MM = 1024
HID = 2048
N_Q = 576
HEADS = 8
HEAD_DIM = 128
BATCH = 4
TOKENS = BATCH * N_Q

_DOT = functools.partial(jnp.dot, preferred_element_type=jnp.float32)


def _dot_t(a, b):
    return jax.lax.dot_general(a, b, (((1,), (1,)), ((), ())),
                               preferred_element_type=jnp.float32)


def _ln(x, g, b, eps=1e-5):
    mu = jnp.mean(x, axis=-1, keepdims=True)
    var = jnp.mean((x - mu) ** 2, axis=-1, keepdims=True)
    return (x - mu) * jax.lax.rsqrt(var + eps) * g + b


def _gelu(h):
    return 0.5 * h * (1.0 + jax.lax.erf(h * 0.7071067811865476))


def _q_kernel(query_ref, pos_ref, g_ref, b_ref, wq_ref, bq_ref, qh_ref):
    qf = _ln(query_ref[...], g_ref[...], b_ref[...]) + pos_ref[...]
    qh_ref[...] = _dot_t(qf, wq_ref[...]) + bq_ref[...]


def _resampler_kernel(qh_ref, x_ref, pos_ref, g_ref, b_ref, wk_ref, wv_ref,
                      bk_ref, bv_ref, ow_ref, ob_ref, o_ref):
    kv = _ln(x_ref[...], g_ref[...], b_ref[...])
    kh = _dot_t(kv + pos_ref[...], wk_ref[...]) + bk_ref[...]
    vh = _dot_t(kv, wv_ref[...]) + bv_ref[...]
    qh = qh_ref[...]
    scale = 1.0 / (HEAD_DIM ** 0.5)
    outs = []
    for h in range(HEADS):
        sl = slice(h * HEAD_DIM, (h + 1) * HEAD_DIM)
        s = _dot_t(qh[:, sl], kh[:, sl]) * scale
        s = s - jnp.max(s, axis=-1, keepdims=True)
        e = jnp.exp(s)
        p = e / jnp.sum(e, axis=-1, keepdims=True)
        outs.append(_DOT(p, vh[:, sl]))
    o = jnp.concatenate(outs, axis=1)
    o_ref[...] = _dot_t(o, ow_ref[...]) + ob_ref[...]


def _h_kernel(x_ref, a_ref, w1_ref, b1_ref, wgt_ref, hm_ref, g_ref):
    x = x_ref[...]
    wgt = wgt_ref[...]
    l0 = jnp.sum(x * wgt[0:1, :], axis=1, keepdims=True)
    l1 = jnp.sum(x * wgt[1:2, :], axis=1, keepdims=True)
    m = jnp.maximum(l0, l1)
    e0 = jnp.exp(l0 - m)
    e1 = jnp.exp(l1 - m)
    s = e0 + e1
    p0 = e0 / s
    p1 = e1 / s
    denom = p0 + p1 + 1e-6
    g0 = p0 / denom
    g1 = p1 / denom
    h0 = _gelu(_DOT(x, w1_ref[...]) + b1_ref[...])
    h1 = _gelu(_DOT(a_ref[...], w1_ref[...]) + b1_ref[...])
    hm_ref[...] = g0 * h0 + g1 * h1
    g_ref[...] = jnp.concatenate([g0, g1], axis=1)


def _out_kernel(hm_ref, w2_ref, b2_ref, g_ref, o_ref):
    g = g_ref[...]
    gsum = g[:, 0:1] + g[:, 1:2]
    o_ref[...] = _DOT(hm_ref[...], w2_ref[...]) + gsum * b2_ref[...]


def kernel(x, proj_w1, proj_b1, proj_w2, proj_b2, query, ln_q_g, ln_q_b,
           ln_kv_g, ln_kv_b, in_proj_w, in_proj_b, out_proj_w, out_proj_b,
           w_gate, pos_embed):
    f32 = jnp.float32
    xf = x.reshape(TOKENS, MM)
    bq, bk, bv = (in_proj_b[:MM][None, :], in_proj_b[MM:2 * MM][None, :],
                  in_proj_b[2 * MM:][None, :])
    ln_q_g2, ln_q_b2 = ln_q_g[None, :], ln_q_b[None, :]
    ln_kv_g2, ln_kv_b2 = ln_kv_g[None, :], ln_kv_b[None, :]
    b1_2 = proj_b1[None, :]
    b2_2 = proj_b2[None, :]
    ob_2 = out_proj_b[None, :]
    wgt = w_gate.T

    qh = pl.pallas_call(
        _q_kernel,
        grid=(1,),
        in_specs=[
            pl.BlockSpec((N_Q, MM), lambda i: (0, 0)),
            pl.BlockSpec((N_Q, MM), lambda i: (0, 0)),
            pl.BlockSpec((1, MM), lambda i: (0, 0)),
            pl.BlockSpec((1, MM), lambda i: (0, 0)),
            pl.BlockSpec((MM, MM), lambda i: (0, 0)),
            pl.BlockSpec((1, MM), lambda i: (0, 0)),
        ],
        out_specs=pl.BlockSpec((N_Q, MM), lambda i: (0, 0)),
        out_shape=jax.ShapeDtypeStruct((N_Q, MM), f32),
    )(query, pos_embed, ln_q_g2, ln_q_b2, in_proj_w, bq)

    attn_out = pl.pallas_call(
        _resampler_kernel,
        grid=(BATCH,),
        in_specs=[
            pl.BlockSpec((N_Q, MM), lambda i: (0, 0)),
            pl.BlockSpec((N_Q, MM), lambda i: (i, 0)),
            pl.BlockSpec((N_Q, MM), lambda i: (0, 0)),
            pl.BlockSpec((1, MM), lambda i: (0, 0)),
            pl.BlockSpec((1, MM), lambda i: (0, 0)),
            pl.BlockSpec((MM, MM), lambda i: (1, 0)),
            pl.BlockSpec((MM, MM), lambda i: (2, 0)),
            pl.BlockSpec((1, MM), lambda i: (0, 0)),
            pl.BlockSpec((1, MM), lambda i: (0, 0)),
            pl.BlockSpec((MM, MM), lambda i: (0, 0)),
            pl.BlockSpec((1, MM), lambda i: (0, 0)),
        ],
        out_specs=pl.BlockSpec((N_Q, MM), lambda i: (i, 0)),
        out_shape=jax.ShapeDtypeStruct((TOKENS, MM), f32),
        compiler_params=pltpu.CompilerParams(
            dimension_semantics=("parallel",)),
    )(qh, xf, pos_embed, ln_kv_g2, ln_kv_b2, in_proj_w, in_proj_w,
      bk, bv, out_proj_w, ob_2)

    MT = 256
    hm, gates = pl.pallas_call(
        _h_kernel,
        grid=(TOKENS // MT,),
        in_specs=[
            pl.BlockSpec((MT, MM), lambda i: (i, 0)),
            pl.BlockSpec((MT, MM), lambda i: (i, 0)),
            pl.BlockSpec((MM, HID), lambda i: (0, 0)),
            pl.BlockSpec((1, HID), lambda i: (0, 0)),
            pl.BlockSpec((2, MM), lambda i: (0, 0)),
        ],
        out_specs=[
            pl.BlockSpec((MT, HID), lambda i: (i, 0)),
            pl.BlockSpec((MT, 2), lambda i: (i, 0)),
        ],
        out_shape=[
            jax.ShapeDtypeStruct((TOKENS, HID), f32),
            jax.ShapeDtypeStruct((TOKENS, 2), f32),
        ],
        compiler_params=pltpu.CompilerParams(
            dimension_semantics=("parallel",)),
    )(xf, attn_out, proj_w1, b1_2, wgt)

    out = pl.pallas_call(
        _out_kernel,
        grid=(TOKENS // MT,),
        in_specs=[
            pl.BlockSpec((MT, HID), lambda i: (i, 0)),
            pl.BlockSpec((HID, HID), lambda i: (0, 0)),
            pl.BlockSpec((1, HID), lambda i: (0, 0)),
            pl.BlockSpec((MT, 2), lambda i: (i, 0)),
        ],
        out_specs=pl.BlockSpec((MT, HID), lambda i: (i, 0)),
        out_shape=jax.ShapeDtypeStruct((TOKENS, HID), f32),
        compiler_params=pltpu.CompilerParams(
            dimension_semantics=("parallel",)),
    )(hm, proj_w2, b2_2, gates)

    return out.reshape(BATCH, N_Q, HID)
